# baseline (speedup 1.0000x reference)
import functools

import jax
import jax.numpy as jnp
from jax import lax
from jax.experimental import pallas as pl
from jax.experimental.pallas import tpu as pltpu
from jax.experimental.pallas import tpu_sc as plsc

S, D_K, D_A, N, B = 4, 64, 1024, 100000, 64
T = 0.07
K_MAX = 64
NB = 4096
GRID = (N + NB - 1) // NB
N_PAD = GRID * NB
NEG = -1e30

NC, NS = 2, 16
ROWS_PER_W = B // (NC * NS)
CH = 10240
NCH = N_PAD // CH
GV = 8
CAP = 512
NEG_F = -3.0e38


def _score_kernel(w_ref, lt_ref, z_ref, wq_ref, pool_ref,
                  s_ref, sume_ref, sumv_ref, qn_ref, acc_e, acc_v):
    step = pl.program_id(0)

    @pl.when(step == 0)
    def _init():
        for s in range(S):
            q = lax.dot_general(z_ref[...], wq_ref[s], (((1,), (1,)), ((), ())),
                                preferred_element_type=jnp.float32)
            nrm = jnp.sqrt(jnp.sum(q * q, axis=1, keepdims=True))
            qn_ref[s] = q / (nrm + 1e-8)
        acc_e[...] = jnp.zeros_like(acc_e)
        acc_v[...] = jnp.zeros_like(acc_v)

    terms = []
    for s in range(S):
        p = pool_ref[s]
        nrm = jnp.sqrt(jnp.sum(p * p, axis=1, keepdims=True))
        pn = p / (nrm + 1e-8)
        sim = lax.dot_general(qn_ref[s], pn, (((1,), (1,)), ((), ())),
                              preferred_element_type=jnp.float32)
        simb = sim.astype(jnp.bfloat16).astype(jnp.float32)
        terms.append(w_ref[s] * simb)
    sblk = (terms[0] + terms[1]) + (terms[2] + terms[3])

    cols = step * NB + lax.broadcasted_iota(jnp.int32, (B, NB), 1)
    valid = cols < N
    e = jnp.exp(sblk * (1.0 / T))
    g = 1.0 / (1.0 + jnp.exp(-lt_ref[0] * (sblk - lt_ref[1])))
    e = jnp.where(valid, e, 0.0)
    v = jnp.where(valid, g * e, 0.0)
    acc_e[...] += jnp.sum(e, axis=1, keepdims=True)
    acc_v[...] += jnp.sum(v, axis=1, keepdims=True)
    s_ref[...] = jnp.where(valid, sblk, NEG)

    @pl.when(step == GRID - 1)
    def _fin():
        sume_ref[...] = acc_e[...]
        sumv_ref[...] = acc_v[...]


def _soft_kernel(s_ref, sume_ref, o_ref):
    o_ref[...] = jnp.exp(s_ref[...] * (1.0 / T)) * (1.0 / sume_ref[...])


def _topk_sc_kernel(s_hbm, stop_hbm, sidx_hbm, buf, cand_v, cand_i, top_v, top_i):
    cid = lax.axis_index("c")
    sid = lax.axis_index("s")
    wid = sid * NC + cid
    lanes = lax.iota(jnp.int32, 16)
    lane0 = lanes == 0

    def extract_and_rebuild(_):
        def ext_body(k, carry):
            def mx_body(i, mx):
                return jnp.maximum(mx, cand_v[pl.ds(i * 16, 16)])
            mx = lax.fori_loop(0, CAP // 16, mx_body,
                               jnp.full((16,), NEG_F, jnp.float32))
            m_val = jnp.max(mx)

            def ps_body(i, pm):
                v = cand_v[pl.ds(i * 16, 16)]
                pos = i * 16 + lanes
                return jnp.minimum(pm, jnp.where(v == m_val, pos, CAP))
            pm = lax.fori_loop(0, CAP // 16, ps_body,
                               jnp.full((16,), CAP, jnp.int32))
            p_vec = jnp.full((16,), jnp.min(pm), jnp.int32)
            idx = jnp.max(plsc.load_gather(cand_i, [p_vec]))
            plsc.store_scatter(cand_v, [p_vec],
                               jnp.full((16,), NEG_F, jnp.float32), mask=lane0)
            k_vec = jnp.full((16,), k, jnp.int32)
            plsc.store_scatter(top_v, [k_vec],
                               jnp.full((16,), m_val, jnp.float32), mask=lane0)
            plsc.store_scatter(top_i, [k_vec],
                               jnp.full((16,), idx, jnp.int32), mask=lane0)
            return carry
        lax.fori_loop(0, K_MAX, ext_body, 0)

        def cp_body(i, c):
            cand_v[pl.ds(i * 16, 16)] = top_v[pl.ds(i * 16, 16)]
            cand_i[pl.ds(i * 16, 16)] = top_i[pl.ds(i * 16, 16)]
            return c
        lax.fori_loop(0, K_MAX // 16, cp_body, 0)

        def cl_body(i, c):
            cand_v[pl.ds(K_MAX + i * 16, 16)] = jnp.full((16,), NEG_F, jnp.float32)
            return c
        lax.fori_loop(0, (CAP - K_MAX) // 16, cl_body, 0)

        def tm_body(i, mn):
            return jnp.minimum(mn, top_v[pl.ds(i * 16, 16)])
        t_new = jnp.min(lax.fori_loop(0, K_MAX // 16, tm_body,
                                      jnp.full((16,), 3.0e38, jnp.float32)))
        return t_new

    for r in range(ROWS_PER_W):
        row = wid * ROWS_PER_W + r

        def clr_body(i, c):
            cand_v[pl.ds(i * 16, 16)] = jnp.full((16,), NEG_F, jnp.float32)
            return c
        lax.fori_loop(0, CAP // 16, clr_body, 0)

        def chunk_body(ch, tc):
            pltpu.sync_copy(s_hbm.at[row, pl.ds(ch * CH, CH)], buf)

            def grp_body(gi, tc):
                t, nc = tc
                base = gi * (16 * GV)
                gmax = jnp.full((16,), NEG_F, jnp.float32)
                for j in range(GV):
                    gmax = jnp.maximum(gmax, buf[pl.ds(base + j * 16, 16)])

                def hit(tc):
                    t, nc = tc
                    for j in range(GV):
                        v = buf[pl.ds(base + j * 16, 16)]
                        mask = v > t
                        csum = jnp.cumsum(mask.astype(jnp.int32))
                        pos = nc + csum - 1
                        gidx = ch * CH + base + j * 16 + lanes
                        plsc.store_scatter(cand_v, [pos], v, mask=mask)
                        plsc.store_scatter(cand_i, [pos], gidx, mask=mask)
                        nc = nc + jnp.max(csum)
                    return lax.cond(nc >= CAP - 16 * GV,
                                    lambda c: (extract_and_rebuild(0),
                                               jnp.int32(K_MAX)),
                                    lambda c: c, (t, nc))

                return lax.cond(jnp.max(gmax) > t, hit, lambda c: c, (t, nc))

            return lax.fori_loop(0, CH // (16 * GV), grp_body, tc)

        lax.fori_loop(0, NCH, chunk_body,
                      (jnp.float32(NEG_F), jnp.int32(0)))
        extract_and_rebuild(0)
        pltpu.sync_copy(top_v, stop_hbm.at[row])
        pltpu.sync_copy(top_i, sidx_hbm.at[row])


def _alpha_kernel(lt_ref, stop_ref, sumv_ref, a_ref):
    s = stop_ref[...]
    e = jnp.exp(s * (1.0 / T))
    g = 1.0 / (1.0 + jnp.exp(-lt_ref[0] * (s - lt_ref[1])))
    traw = g * e / (sumv_ref[...] + 1e-8)
    a_ref[...] = traw / (jnp.sum(traw, axis=1, keepdims=True) + 1e-8)


def kernel(z, pool_keys, W_Q, aspect_weights, tau, centroids, lambda_val, is_warmup):
    del centroids, is_warmup
    w = jax.nn.softmax(aspect_weights.astype(jnp.float32), axis=0)
    w = w.astype(jnp.bfloat16).astype(jnp.float32)
    lt = jnp.stack([jnp.asarray(lambda_val, jnp.float32),
                    jnp.asarray(tau, jnp.float32)])

    s_pad, sum_e, sum_v = pl.pallas_call(
        _score_kernel,
        grid=(GRID,),
        in_specs=[
            pl.BlockSpec(memory_space=pltpu.SMEM),
            pl.BlockSpec(memory_space=pltpu.SMEM),
            pl.BlockSpec((B, D_A), lambda i: (0, 0)),
            pl.BlockSpec((S, D_K, D_A), lambda i: (0, 0, 0)),
            pl.BlockSpec((S, NB, D_K), lambda i: (0, i, 0)),
        ],
        out_specs=[
            pl.BlockSpec((B, NB), lambda i: (0, i)),
            pl.BlockSpec((B, 1), lambda i: (0, 0)),
            pl.BlockSpec((B, 1), lambda i: (0, 0)),
        ],
        out_shape=[
            jax.ShapeDtypeStruct((B, N_PAD), jnp.float32),
            jax.ShapeDtypeStruct((B, 1), jnp.float32),
            jax.ShapeDtypeStruct((B, 1), jnp.float32),
        ],
        scratch_shapes=[
            pltpu.VMEM((S, B, D_K), jnp.float32),
            pltpu.VMEM((B, 1), jnp.float32),
            pltpu.VMEM((B, 1), jnp.float32),
        ],
    )(w, lt, z, W_Q, pool_keys)

    soft_full = pl.pallas_call(
        _soft_kernel,
        grid=(GRID,),
        in_specs=[
            pl.BlockSpec((B, NB), lambda i: (0, i)),
            pl.BlockSpec((B, 1), lambda i: (0, 0)),
        ],
        out_specs=pl.BlockSpec((B, NB), lambda i: (0, i)),
        out_shape=jax.ShapeDtypeStruct((B, N), jnp.float32),
    )(s_pad, sum_e)

    s_top, idx_top = pl.kernel(
        _topk_sc_kernel,
        out_type=[jax.ShapeDtypeStruct((B, K_MAX), jnp.float32),
                  jax.ShapeDtypeStruct((B, K_MAX), jnp.int32)],
        mesh=plsc.VectorSubcoreMesh(core_axis_name="c", subcore_axis_name="s"),
        compiler_params=pltpu.CompilerParams(needs_layout_passes=False),
        scratch_types=[
            pltpu.VMEM((CH,), jnp.float32),
            pltpu.VMEM((CAP,), jnp.float32),
            pltpu.VMEM((CAP,), jnp.int32),
            pltpu.VMEM((K_MAX,), jnp.float32),
            pltpu.VMEM((K_MAX,), jnp.int32),
        ],
    )(s_pad)

    alphas = pl.pallas_call(
        _alpha_kernel,
        in_specs=[
            pl.BlockSpec(memory_space=pltpu.SMEM),
            pl.BlockSpec((B, K_MAX), lambda: (0, 0)),
            pl.BlockSpec((B, 1), lambda: (0, 0)),
        ],
        out_specs=pl.BlockSpec((B, K_MAX), lambda: (0, 0)),
        out_shape=jax.ShapeDtypeStruct((B, K_MAX), jnp.float32),
    )(lt, s_top, sum_v)

    return (alphas, idx_top.astype(jnp.int32), soft_full)
